# baseline (speedup 1.0000x reference)
import math

import jax
import jax.numpy as jnp
from jax import lax
from jax.experimental import pallas as pl
from jax.experimental.pallas import tpu as pltpu
from jax.experimental.pallas import tpu_sc as plsc

NUM_USER = 1000000
NUM_ITEM = 1000000
F = 64
B = 4096
REG = 0.01

NC = 2
NS = 16
NW = NC * NS
BPW = B // NW
L = 16

_INV_LN2 = 1.0 / math.log(2.0)


def _sc_gather3_t(u, i, j, user_flat, item_flat):
    mesh = plsc.VectorSubcoreMesh(core_axis_name="c", subcore_axis_name="s")
    out_t = jax.ShapeDtypeStruct((F, B), jnp.float32)

    @pl.kernel(
        out_type=[out_t, out_t, out_t],
        mesh=mesh,
        scratch_types=[
            pltpu.VMEM((BPW,), jnp.int32),
            pltpu.VMEM((BPW,), jnp.int32),
            pltpu.VMEM((BPW,), jnp.int32),
            pltpu.VMEM((F, BPW), jnp.int32),
            pltpu.VMEM((F, BPW), jnp.int32),
            pltpu.VMEM((F, BPW), jnp.int32),
            pltpu.VMEM((F, BPW), jnp.float32),
            pltpu.VMEM((F, BPW), jnp.float32),
            pltpu.VMEM((F, BPW), jnp.float32),
            pltpu.SemaphoreType.DMA,
            pltpu.SemaphoreType.DMA,
            pltpu.SemaphoreType.DMA,
        ],
    )
    def gather_kernel(u_hbm, i_hbm, j_hbm, uflat_hbm, iflat_hbm,
                      eut_hbm, eit_hbm, ejt_hbm,
                      idx_u, idx_i, idx_j, fidx_u, fidx_i, fidx_j,
                      gu, gi, gj, sem_u, sem_i, sem_j):
        wid = lax.axis_index("s") * NC + lax.axis_index("c")
        sl = pl.ds(wid * BPW, BPW)
        pltpu.sync_copy(u_hbm.at[sl], idx_u)
        pltpu.sync_copy(i_hbm.at[sl], idx_i)
        pltpu.sync_copy(j_hbm.at[sl], idx_j)

        @pl.loop(0, F)
        def _(f):
            base_u = f * NUM_USER
            base_i = f * NUM_ITEM
            for c in range(BPW // L):
                cs = pl.ds(c * L, L)
                idx_c_u = idx_u[cs]
                idx_c_i = idx_i[cs]
                idx_c_j = idx_j[cs]
                fidx_u[f, cs] = idx_c_u + base_u
                fidx_i[f, cs] = idx_c_i + base_i
                fidx_j[f, cs] = idx_c_j + base_i

        @pl.loop(0, F)
        def _(f):
            pltpu.async_copy(uflat_hbm.at[fidx_u.at[f]], gu.at[f], sem_u)
            pltpu.async_copy(iflat_hbm.at[fidx_i.at[f]], gi.at[f], sem_i)
            pltpu.async_copy(iflat_hbm.at[fidx_j.at[f]], gj.at[f], sem_j)

        pltpu.make_async_copy(eut_hbm.at[:, sl], gu, sem_u).wait()
        pltpu.make_async_copy(eit_hbm.at[:, sl], gi, sem_i).wait()
        pltpu.make_async_copy(ejt_hbm.at[:, sl], gj, sem_j).wait()

        pltpu.sync_copy(gu, eut_hbm.at[:, sl])
        pltpu.sync_copy(gi, eit_hbm.at[:, sl])
        pltpu.sync_copy(gj, ejt_hbm.at[:, sl])

    return gather_kernel(u, i, j, user_flat, item_flat)


def _tc_body(eut_ref, eit_ref, ejt_ref, yui_ref, yuj_ref, loss_ref):
    eu = eut_ref[...]
    ei = eit_ref[...]
    ej = ejt_ref[...]
    s_i = jnp.sum(ei, axis=1, keepdims=True)
    s_j = jnp.sum(ej, axis=1, keepdims=True)
    y_ui = jnp.sum(eu * s_i, axis=0)
    y_uj = jnp.sum(eu * s_j, axis=0)
    yui_ref[...] = y_ui
    yuj_ref[...] = y_uj
    reg = REG * (jnp.sum(eu * eu) + jnp.sum(ei * ei) + jnp.sum(ej * ej))
    d = y_ui - y_uj
    ls = jnp.minimum(d, 0.0) - jnp.log(1.0 + jnp.exp(-jnp.abs(d)))
    loss_ref[0, 0] = reg - jnp.sum(ls) * _INV_LN2


def _tc_finish(eut, eit, ejt):
    return pl.pallas_call(
        _tc_body,
        out_shape=(
            jax.ShapeDtypeStruct((B,), jnp.float32),
            jax.ShapeDtypeStruct((B,), jnp.float32),
            jax.ShapeDtypeStruct((1, 1), jnp.float32),
        ),
        out_specs=(
            pl.BlockSpec(memory_space=pltpu.VMEM),
            pl.BlockSpec(memory_space=pltpu.VMEM),
            pl.BlockSpec(memory_space=pltpu.SMEM),
        ),
    )(eut, eit, ejt)


def kernel(u, i, j, embed_user, embed_item):
    user_flat = embed_user.T.reshape(NUM_USER * F)
    item_flat = embed_item.T.reshape(NUM_ITEM * F)
    eut, eit, ejt = _sc_gather3_t(u, i, j, user_flat, item_flat)
    y_ui, y_uj, loss2d = _tc_finish(eut, eit, ejt)
    return y_ui, y_uj, loss2d[0, 0]


# speedup vs baseline: 12.3035x; 12.3035x over previous
import functools
import math

import jax
import jax.numpy as jnp
from jax import lax
from jax.experimental import pallas as pl
from jax.experimental.pallas import tpu as pltpu
from jax.experimental.pallas import tpu_sc as plsc

V = 1000000
F = 64
B = 4096
REG = 0.01

NC = 2
NS = 16
NW = NC * NS
BPW = B // NW

BLK = 16384
GRID = (V + BLK - 1) // BLK

ZCH = 16384
WSPAN = V // 8

_INV_LN2 = 1.0 / math.log(2.0)


def _sc_counts(i, j):
    mesh = plsc.VectorSubcoreMesh(core_axis_name="c", subcore_axis_name="s")
    cnt = jax.ShapeDtypeStruct((V,), jnp.float32)

    @pl.kernel(
        out_type=[cnt, cnt],
        mesh=mesh,
        compiler_params=pltpu.CompilerParams(use_tc_tiling_on_sc=False),
        scratch_types=[
            pltpu.VMEM((2, BPW), jnp.int32),
            pltpu.VMEM((BPW,), jnp.float32),
            pltpu.VMEM((ZCH,), jnp.float32),
            pltpu.VMEM_SHARED((V,), jnp.float32),
            pltpu.SemaphoreType.DMA,
        ],
    )
    def counts_kernel(i_hbm, j_hbm, ci_hbm, cj_hbm, idx2, ones, zbuf, csh, sem):
        c = lax.axis_index("c")
        s = lax.axis_index("s")

        @pl.loop(0, BPW, step=16)
        def _(k):
            ones[pl.ds(k, 16)] = jnp.full((16,), 1.0, jnp.float32)

        @pl.loop(0, ZCH, step=16)
        def _(k):
            zbuf[pl.ds(k, 16)] = jnp.zeros((16,), jnp.float32)

        @pl.when(s < 8)
        def _():
            n_full = WSPAN // ZCH
            tail = WSPAN - n_full * ZCH
            cps = []
            for kk in range(n_full):
                cps.append(pltpu.async_copy(
                    zbuf, csh.at[pl.ds(s * WSPAN + kk * ZCH, ZCH)], sem))
            cps.append(pltpu.async_copy(
                zbuf.at[pl.ds(0, tail)],
                csh.at[pl.ds(s * WSPAN + n_full * ZCH, tail)], sem))
            for cp in cps:
                cp.wait()

        plsc.subcore_barrier()

        @pl.when(c == 0)
        def _():
            pltpu.sync_copy(i_hbm.at[pl.ds(s * 256, BPW)], idx2.at[0])
            pltpu.sync_copy(i_hbm.at[pl.ds(s * 256 + BPW, BPW)], idx2.at[1])

        @pl.when(c == 1)
        def _():
            pltpu.sync_copy(j_hbm.at[pl.ds(s * 256, BPW)], idx2.at[0])
            pltpu.sync_copy(j_hbm.at[pl.ds(s * 256 + BPW, BPW)], idx2.at[1])

        pltpu.sync_copy(ones, csh.at[idx2.at[0]], add=True)
        pltpu.sync_copy(ones, csh.at[idx2.at[1]], add=True)

        plsc.subcore_barrier()

        @pl.when(jnp.logical_and(s < 8, c == 0))
        def _():
            pltpu.sync_copy(csh.at[pl.ds(s * WSPAN, WSPAN)],
                            ci_hbm.at[pl.ds(s * WSPAN, WSPAN)])

        @pl.when(jnp.logical_and(s < 8, c == 1))
        def _():
            pltpu.sync_copy(csh.at[pl.ds(s * WSPAN, WSPAN)],
                            cj_hbm.at[pl.ds(s * WSPAN, WSPAN)])

    return counts_kernel(i, j)


def _tc_item_body(it_ref, ci_ref, cj_ref, st_ref, q_ref):
    n = pl.program_id(0)
    e = it_ref[...]
    ci = ci_ref[...]
    cj = cj_ref[...]
    pos = n * BLK + lax.broadcasted_iota(jnp.int32, (BLK,), 0)
    valid = pos < V
    e = jnp.where(valid[None, :], e, 0.0)
    ci = jnp.where(valid, ci, 0.0)
    cj = jnp.where(valid, cj, 0.0)
    q_ref[...] = jnp.sum(e * e, axis=0)
    c2 = jnp.concatenate([ci[None, :], cj[None, :]], axis=0)
    part = lax.dot_general(e, c2, (((1,), (1,)), ((), ())),
                           precision=lax.Precision.HIGHEST,
                           preferred_element_type=jnp.float32)

    @pl.when(n == 0)
    def _():
        st_ref[...] = jnp.zeros((F, 2), jnp.float32)

    st_ref[...] += part


def _tc_item(item_t, ci, cj):
    return pl.pallas_call(
        _tc_item_body,
        grid=(GRID,),
        in_specs=[
            pl.BlockSpec((F, BLK), lambda n: (0, n)),
            pl.BlockSpec((BLK,), lambda n: (n,)),
            pl.BlockSpec((BLK,), lambda n: (n,)),
        ],
        out_specs=(
            pl.BlockSpec((F, 2), lambda n: (0, 0)),
            pl.BlockSpec((BLK,), lambda n: (n,)),
        ),
        out_shape=(
            jax.ShapeDtypeStruct((F, 2), jnp.float32),
            jax.ShapeDtypeStruct((V,), jnp.float32),
        ),
    )(item_t, ci, cj)


def _tc_user_body(ut_ref, st_ref, zi_ref, zj_ref, q_ref):
    e = ut_ref[...]
    w = st_ref[...]
    z = lax.dot_general(w, e, (((0,), (0,)), ((), ())),
                        precision=lax.Precision.HIGHEST,
                        preferred_element_type=jnp.float32)
    zi_ref[...] = z[0, :]
    zj_ref[...] = z[1, :]
    q_ref[...] = jnp.sum(e * e, axis=0)


def _tc_user(user_t, st):
    return pl.pallas_call(
        _tc_user_body,
        grid=(GRID,),
        in_specs=[
            pl.BlockSpec((F, BLK), lambda n: (0, n)),
            pl.BlockSpec((F, 2), lambda n: (0, 0)),
        ],
        out_specs=(
            pl.BlockSpec((BLK,), lambda n: (n,)),
            pl.BlockSpec((BLK,), lambda n: (n,)),
            pl.BlockSpec((BLK,), lambda n: (n,)),
        ),
        out_shape=(
            jax.ShapeDtypeStruct((V,), jnp.float32),
            jax.ShapeDtypeStruct((V,), jnp.float32),
            jax.ShapeDtypeStruct((V,), jnp.float32),
        ),
    )(user_t, st)


def _sc_gather5(u, i, j, zi, zj, qu, qi):
    mesh = plsc.VectorSubcoreMesh(core_axis_name="c", subcore_axis_name="s")
    vec = jax.ShapeDtypeStruct((B,), jnp.float32)

    @pl.kernel(
        out_type=[vec, vec, vec, vec, vec],
        mesh=mesh,
        compiler_params=pltpu.CompilerParams(use_tc_tiling_on_sc=False),
        scratch_types=[
            pltpu.VMEM((BPW,), jnp.int32),
            pltpu.VMEM((BPW,), jnp.int32),
            pltpu.VMEM((BPW,), jnp.int32),
            pltpu.VMEM((BPW,), jnp.float32),
            pltpu.VMEM((BPW,), jnp.float32),
            pltpu.VMEM((BPW,), jnp.float32),
            pltpu.VMEM((BPW,), jnp.float32),
            pltpu.VMEM((BPW,), jnp.float32),
            pltpu.SemaphoreType.DMA,
        ],
    )
    def gather_kernel(u_hbm, i_hbm, j_hbm, zi_hbm, zj_hbm, qu_hbm, qi_hbm,
                      yui_hbm, yuj_hbm, gqu_hbm, gqi_hbm, gqj_hbm,
                      idx_u, idx_i, idx_j, b0, b1, b2, b3, b4, sem):
        wid = lax.axis_index("s") * NC + lax.axis_index("c")
        sl = pl.ds(wid * BPW, BPW)
        pltpu.sync_copy(u_hbm.at[sl], idx_u)
        pltpu.sync_copy(i_hbm.at[sl], idx_i)
        pltpu.sync_copy(j_hbm.at[sl], idx_j)
        cps = [
            pltpu.async_copy(zi_hbm.at[idx_u], b0, sem),
            pltpu.async_copy(zj_hbm.at[idx_u], b1, sem),
            pltpu.async_copy(qu_hbm.at[idx_u], b2, sem),
            pltpu.async_copy(qi_hbm.at[idx_i], b3, sem),
            pltpu.async_copy(qi_hbm.at[idx_j], b4, sem),
        ]
        for cp in cps:
            cp.wait()
        pltpu.sync_copy(b0, yui_hbm.at[sl])
        pltpu.sync_copy(b1, yuj_hbm.at[sl])
        pltpu.sync_copy(b2, gqu_hbm.at[sl])
        pltpu.sync_copy(b3, gqi_hbm.at[sl])
        pltpu.sync_copy(b4, gqj_hbm.at[sl])

    return gather_kernel(u, i, j, zi, zj, qu, qi)


def _tc_loss_body(yui_ref, yuj_ref, gqu_ref, gqi_ref, gqj_ref, loss_ref):
    y_ui = yui_ref[...]
    y_uj = yuj_ref[...]
    reg = REG * (jnp.sum(gqu_ref[...]) + jnp.sum(gqi_ref[...])
                 + jnp.sum(gqj_ref[...]))
    d = y_ui - y_uj
    ls = jnp.minimum(d, 0.0) - jnp.log(1.0 + jnp.exp(-jnp.abs(d)))
    loss_ref[0, 0] = reg - jnp.sum(ls) * _INV_LN2


def _tc_loss(y_ui, y_uj, gqu, gqi, gqj):
    return pl.pallas_call(
        _tc_loss_body,
        out_shape=jax.ShapeDtypeStruct((1, 1), jnp.float32),
        out_specs=pl.BlockSpec(memory_space=pltpu.SMEM),
    )(y_ui, y_uj, gqu, gqi, gqj)


def kernel(u, i, j, embed_user, embed_item):
    user_t = embed_user.T
    item_t = embed_item.T
    ci, cj = _sc_counts(i, j)
    st, q_item = _tc_item(item_t, ci, cj)
    z_ui, z_uj, q_user = _tc_user(user_t, st)
    y_ui, y_uj, gqu, gqi, gqj = _sc_gather5(u, i, j, z_ui, z_uj, q_user, q_item)
    loss2d = _tc_loss(y_ui, y_uj, gqu, gqi, gqj)
    return y_ui, y_uj, loss2d[0, 0]


# speedup vs baseline: 33.5263x; 2.7250x over previous
import functools
import math

import jax
import jax.numpy as jnp
from jax import lax
from jax.experimental import pallas as pl
from jax.experimental.pallas import tpu as pltpu
from jax.experimental.pallas import tpu_sc as plsc

V = 1000000
F = 64
B = 4096
REG = 0.01

NC = 2
NS = 16
NW = NC * NS
BPW = B // NW

BLK = 32768
GRID = (V + BLK - 1) // BLK

ZCH = 16384
WSPAN = V // 8

_INV_LN2 = 1.0 / math.log(2.0)


def _sc_counts(i, j):
    mesh = plsc.VectorSubcoreMesh(core_axis_name="c", subcore_axis_name="s")
    cnt = jax.ShapeDtypeStruct((V,), jnp.float32)

    @pl.kernel(
        out_type=[cnt, cnt],
        mesh=mesh,
        compiler_params=pltpu.CompilerParams(use_tc_tiling_on_sc=False),
        scratch_types=[
            pltpu.VMEM((2, BPW), jnp.int32),
            pltpu.VMEM((BPW,), jnp.float32),
            pltpu.VMEM((ZCH,), jnp.float32),
            pltpu.VMEM_SHARED((V,), jnp.float32),
            pltpu.SemaphoreType.DMA,
        ],
    )
    def counts_kernel(i_hbm, j_hbm, ci_hbm, cj_hbm, idx2, ones, zbuf, csh, sem):
        c = lax.axis_index("c")
        s = lax.axis_index("s")

        @pl.loop(0, BPW, step=16)
        def _(k):
            ones[pl.ds(k, 16)] = jnp.full((16,), 1.0, jnp.float32)

        @pl.loop(0, ZCH, step=16)
        def _(k):
            zbuf[pl.ds(k, 16)] = jnp.zeros((16,), jnp.float32)

        @pl.when(s < 8)
        def _():
            n_full = WSPAN // ZCH
            tail = WSPAN - n_full * ZCH
            cps = []
            for kk in range(n_full):
                cps.append(pltpu.async_copy(
                    zbuf, csh.at[pl.ds(s * WSPAN + kk * ZCH, ZCH)], sem))
            cps.append(pltpu.async_copy(
                zbuf.at[pl.ds(0, tail)],
                csh.at[pl.ds(s * WSPAN + n_full * ZCH, tail)], sem))
            for cp in cps:
                cp.wait()

        plsc.subcore_barrier()

        @pl.when(c == 0)
        def _():
            pltpu.sync_copy(i_hbm.at[pl.ds(s * 256, BPW)], idx2.at[0])
            pltpu.sync_copy(i_hbm.at[pl.ds(s * 256 + BPW, BPW)], idx2.at[1])

        @pl.when(c == 1)
        def _():
            pltpu.sync_copy(j_hbm.at[pl.ds(s * 256, BPW)], idx2.at[0])
            pltpu.sync_copy(j_hbm.at[pl.ds(s * 256 + BPW, BPW)], idx2.at[1])

        pltpu.sync_copy(ones, csh.at[idx2.at[0]], add=True)
        pltpu.sync_copy(ones, csh.at[idx2.at[1]], add=True)

        plsc.subcore_barrier()

        @pl.when(jnp.logical_and(s < 8, c == 0))
        def _():
            pltpu.sync_copy(csh.at[pl.ds(s * WSPAN, WSPAN)],
                            ci_hbm.at[pl.ds(s * WSPAN, WSPAN)])

        @pl.when(jnp.logical_and(s < 8, c == 1))
        def _():
            pltpu.sync_copy(csh.at[pl.ds(s * WSPAN, WSPAN)],
                            cj_hbm.at[pl.ds(s * WSPAN, WSPAN)])

    return counts_kernel(i, j)


def _tc_item_body(it_ref, ci_ref, cj_ref, st_ref, q_ref):
    n = pl.program_id(0)
    e = it_ref[...]
    ci = ci_ref[...]
    cj = cj_ref[...]

    @pl.when(n == 0)
    def _():
        st_ref[...] = jnp.zeros((2, F), jnp.float32)

    q_ref[...] = jnp.sum(e * e, axis=0)

    def accum(e_, ci_, cj_):
        si = jnp.sum(e_ * ci_[None, :], axis=1)
        sj = jnp.sum(e_ * cj_[None, :], axis=1)
        st_ref[0:1, :] += si[None, :]
        st_ref[1:2, :] += sj[None, :]

    @pl.when(n < GRID - 1)
    def _():
        accum(e, ci, cj)

    @pl.when(n == GRID - 1)
    def _():
        pos = n * BLK + lax.broadcasted_iota(jnp.int32, (BLK,), 0)
        valid = pos < V
        accum(jnp.where(valid[None, :], e, 0.0),
              jnp.where(valid, ci, 0.0), jnp.where(valid, cj, 0.0))


def _tc_item(item_t, ci, cj):
    return pl.pallas_call(
        _tc_item_body,
        grid=(GRID,),
        in_specs=[
            pl.BlockSpec((F, BLK), lambda n: (0, n)),
            pl.BlockSpec((BLK,), lambda n: (n,)),
            pl.BlockSpec((BLK,), lambda n: (n,)),
        ],
        out_specs=(
            pl.BlockSpec((2, F), lambda n: (0, 0)),
            pl.BlockSpec((BLK,), lambda n: (n,)),
        ),
        out_shape=(
            jax.ShapeDtypeStruct((2, F), jnp.float32),
            jax.ShapeDtypeStruct((V,), jnp.float32),
        ),
    )(item_t, ci, cj)


def _tc_user_body(ut_ref, st_ref, zi_ref, zj_ref, q_ref):
    e = ut_ref[...]
    w = st_ref[...]
    z = lax.dot_general(w, e, (((1,), (0,)), ((), ())),
                        precision=lax.Precision.HIGHEST,
                        preferred_element_type=jnp.float32)
    zi_ref[...] = z[0, :]
    zj_ref[...] = z[1, :]
    q_ref[...] = jnp.sum(e * e, axis=0)


def _tc_user(user_t, st):
    return pl.pallas_call(
        _tc_user_body,
        grid=(GRID,),
        in_specs=[
            pl.BlockSpec((F, BLK), lambda n: (0, n)),
            pl.BlockSpec((2, F), lambda n: (0, 0)),
        ],
        out_specs=(
            pl.BlockSpec((BLK,), lambda n: (n,)),
            pl.BlockSpec((BLK,), lambda n: (n,)),
            pl.BlockSpec((BLK,), lambda n: (n,)),
        ),
        out_shape=(
            jax.ShapeDtypeStruct((V,), jnp.float32),
            jax.ShapeDtypeStruct((V,), jnp.float32),
            jax.ShapeDtypeStruct((V,), jnp.float32),
        ),
    )(user_t, st)


def _sc_gather5(u, i, j, zi, zj, qu, qi):
    mesh = plsc.VectorSubcoreMesh(core_axis_name="c", subcore_axis_name="s")
    vec = jax.ShapeDtypeStruct((B,), jnp.float32)

    @pl.kernel(
        out_type=[vec, vec, vec, vec, vec],
        mesh=mesh,
        compiler_params=pltpu.CompilerParams(use_tc_tiling_on_sc=False),
        scratch_types=[
            pltpu.VMEM((BPW,), jnp.int32),
            pltpu.VMEM((BPW,), jnp.int32),
            pltpu.VMEM((BPW,), jnp.int32),
            pltpu.VMEM((BPW,), jnp.float32),
            pltpu.VMEM((BPW,), jnp.float32),
            pltpu.VMEM((BPW,), jnp.float32),
            pltpu.VMEM((BPW,), jnp.float32),
            pltpu.VMEM((BPW,), jnp.float32),
            pltpu.SemaphoreType.DMA,
        ],
    )
    def gather_kernel(u_hbm, i_hbm, j_hbm, zi_hbm, zj_hbm, qu_hbm, qi_hbm,
                      yui_hbm, yuj_hbm, gqu_hbm, gqi_hbm, gqj_hbm,
                      idx_u, idx_i, idx_j, b0, b1, b2, b3, b4, sem):
        wid = lax.axis_index("s") * NC + lax.axis_index("c")
        sl = pl.ds(wid * BPW, BPW)
        pltpu.sync_copy(u_hbm.at[sl], idx_u)
        pltpu.sync_copy(i_hbm.at[sl], idx_i)
        pltpu.sync_copy(j_hbm.at[sl], idx_j)
        cps = [
            pltpu.async_copy(zi_hbm.at[idx_u], b0, sem),
            pltpu.async_copy(zj_hbm.at[idx_u], b1, sem),
            pltpu.async_copy(qu_hbm.at[idx_u], b2, sem),
            pltpu.async_copy(qi_hbm.at[idx_i], b3, sem),
            pltpu.async_copy(qi_hbm.at[idx_j], b4, sem),
        ]
        for cp in cps:
            cp.wait()
        pltpu.sync_copy(b0, yui_hbm.at[sl])
        pltpu.sync_copy(b1, yuj_hbm.at[sl])
        pltpu.sync_copy(b2, gqu_hbm.at[sl])
        pltpu.sync_copy(b3, gqi_hbm.at[sl])
        pltpu.sync_copy(b4, gqj_hbm.at[sl])

    return gather_kernel(u, i, j, zi, zj, qu, qi)


def _tc_loss_body(yui_ref, yuj_ref, gqu_ref, gqi_ref, gqj_ref, loss_ref):
    y_ui = yui_ref[...]
    y_uj = yuj_ref[...]
    reg = REG * (jnp.sum(gqu_ref[...]) + jnp.sum(gqi_ref[...])
                 + jnp.sum(gqj_ref[...]))
    d = y_ui - y_uj
    ls = jnp.minimum(d, 0.0) - jnp.log(1.0 + jnp.exp(-jnp.abs(d)))
    loss_ref[0, 0] = reg - jnp.sum(ls) * _INV_LN2


def _tc_loss(y_ui, y_uj, gqu, gqi, gqj):
    return pl.pallas_call(
        _tc_loss_body,
        out_shape=jax.ShapeDtypeStruct((1, 1), jnp.float32),
        out_specs=pl.BlockSpec(memory_space=pltpu.SMEM),
    )(y_ui, y_uj, gqu, gqi, gqj)


def kernel(u, i, j, embed_user, embed_item):
    user_t = embed_user.T
    item_t = embed_item.T
    ci, cj = _sc_counts(i, j)
    st, q_item = _tc_item(item_t, ci, cj)
    z_ui, z_uj, q_user = _tc_user(user_t, st)
    y_ui, y_uj, gqu, gqi, gqj = _sc_gather5(u, i, j, z_ui, z_uj, q_user, q_item)
    loss2d = _tc_loss(y_ui, y_uj, gqu, gqi, gqj)
    return y_ui, y_uj, loss2d[0, 0]


# speedup vs baseline: 34.9932x; 1.0438x over previous
import functools
import math

import jax
import jax.numpy as jnp
from jax import lax
from jax.experimental import pallas as pl
from jax.experimental.pallas import tpu as pltpu
from jax.experimental.pallas import tpu_sc as plsc

V = 1000000
F = 64
B = 4096
REG = 0.01

NC = 2
NS = 16
NW = NC * NS
BPW = B // NW

BLK = 32768
GRID = (V + BLK - 1) // BLK

ZCH = 16384
WSPAN = V // 8

_INV_LN2 = 1.0 / math.log(2.0)


def _sc_counts(i, j):
    mesh = plsc.VectorSubcoreMesh(core_axis_name="c", subcore_axis_name="s")
    cnt = jax.ShapeDtypeStruct((V,), jnp.float32)

    @pl.kernel(
        out_type=[cnt, cnt],
        mesh=mesh,
        compiler_params=pltpu.CompilerParams(use_tc_tiling_on_sc=False),
        scratch_types=[
            pltpu.VMEM((2, BPW), jnp.int32),
            pltpu.VMEM((BPW,), jnp.float32),
            pltpu.VMEM((ZCH,), jnp.float32),
            pltpu.VMEM_SHARED((V,), jnp.float32),
            pltpu.SemaphoreType.DMA,
        ],
    )
    def counts_kernel(i_hbm, j_hbm, ci_hbm, cj_hbm, idx2, ones, zbuf, csh, sem):
        c = lax.axis_index("c")
        s = lax.axis_index("s")

        @pl.loop(0, BPW, step=16)
        def _(k):
            ones[pl.ds(k, 16)] = jnp.full((16,), 1.0, jnp.float32)

        @pl.loop(0, ZCH, step=16)
        def _(k):
            zbuf[pl.ds(k, 16)] = jnp.zeros((16,), jnp.float32)

        @pl.when(s < 8)
        def _():
            n_full = WSPAN // ZCH
            tail = WSPAN - n_full * ZCH
            cps = []
            for kk in range(n_full):
                cps.append(pltpu.async_copy(
                    zbuf, csh.at[pl.ds(s * WSPAN + kk * ZCH, ZCH)], sem))
            cps.append(pltpu.async_copy(
                zbuf.at[pl.ds(0, tail)],
                csh.at[pl.ds(s * WSPAN + n_full * ZCH, tail)], sem))
            for cp in cps:
                cp.wait()

        plsc.subcore_barrier()

        @pl.when(c == 0)
        def _():
            pltpu.sync_copy(i_hbm.at[pl.ds(s * 256, BPW)], idx2.at[0])
            pltpu.sync_copy(i_hbm.at[pl.ds(s * 256 + BPW, BPW)], idx2.at[1])

        @pl.when(c == 1)
        def _():
            pltpu.sync_copy(j_hbm.at[pl.ds(s * 256, BPW)], idx2.at[0])
            pltpu.sync_copy(j_hbm.at[pl.ds(s * 256 + BPW, BPW)], idx2.at[1])

        pltpu.sync_copy(ones, csh.at[idx2.at[0]], add=True)
        pltpu.sync_copy(ones, csh.at[idx2.at[1]], add=True)

        plsc.subcore_barrier()

        @pl.when(jnp.logical_and(s < 8, c == 0))
        def _():
            pltpu.sync_copy(csh.at[pl.ds(s * WSPAN, WSPAN)],
                            ci_hbm.at[pl.ds(s * WSPAN, WSPAN)])

        @pl.when(jnp.logical_and(s < 8, c == 1))
        def _():
            pltpu.sync_copy(csh.at[pl.ds(s * WSPAN, WSPAN)],
                            cj_hbm.at[pl.ds(s * WSPAN, WSPAN)])

    return counts_kernel(i, j)


def _tc_item_body(it_ref, ci_ref, cj_ref, st_ref, q_ref, acc_ref):
    n = pl.program_id(0)
    e = it_ref[...]
    ci = ci_ref[...]
    cj = cj_ref[...]

    @pl.when(n == 0)
    def _():
        acc_ref[...] = jnp.zeros((2, F, 128), jnp.float32)

    q_ref[...] = jnp.sum(e * e, axis=0)

    def accum(e_, ci_, cj_):
        ai = acc_ref[0]
        aj = acc_ref[1]
        cib = ci_[None, :]
        cjb = cj_[None, :]
        for k in range(BLK // 128):
            s = slice(k * 128, (k + 1) * 128)
            ec = e_[:, s]
            ai = ai + ec * cib[:, s]
            aj = aj + ec * cjb[:, s]
        acc_ref[0] = ai
        acc_ref[1] = aj

    @pl.when(n < GRID - 1)
    def _():
        accum(e, ci, cj)

    @pl.when(n == GRID - 1)
    def _():
        pos = n * BLK + lax.broadcasted_iota(jnp.int32, (BLK,), 0)
        valid = pos < V
        accum(jnp.where(valid[None, :], e, 0.0),
              jnp.where(valid, ci, 0.0), jnp.where(valid, cj, 0.0))
        st_ref[0:1, :] = jnp.sum(acc_ref[0], axis=1)[None, :]
        st_ref[1:2, :] = jnp.sum(acc_ref[1], axis=1)[None, :]


def _tc_item(item_t, ci, cj):
    return pl.pallas_call(
        _tc_item_body,
        grid=(GRID,),
        in_specs=[
            pl.BlockSpec((F, BLK), lambda n: (0, n)),
            pl.BlockSpec((BLK,), lambda n: (n,)),
            pl.BlockSpec((BLK,), lambda n: (n,)),
        ],
        out_specs=(
            pl.BlockSpec((2, F), lambda n: (0, 0)),
            pl.BlockSpec((BLK,), lambda n: (n,)),
        ),
        out_shape=(
            jax.ShapeDtypeStruct((2, F), jnp.float32),
            jax.ShapeDtypeStruct((V,), jnp.float32),
        ),
        scratch_shapes=[pltpu.VMEM((2, F, 128), jnp.float32)],
    )(item_t, ci, cj)


def _tc_user_body(ut_ref, st_ref, zi_ref, zj_ref, q_ref):
    e = ut_ref[...]
    w = st_ref[...]
    z = lax.dot_general(w, e, (((1,), (0,)), ((), ())),
                        precision=lax.Precision.HIGHEST,
                        preferred_element_type=jnp.float32)
    zi_ref[...] = z[0, :]
    zj_ref[...] = z[1, :]
    q_ref[...] = jnp.sum(e * e, axis=0)


def _tc_user(user_t, st):
    return pl.pallas_call(
        _tc_user_body,
        grid=(GRID,),
        in_specs=[
            pl.BlockSpec((F, BLK), lambda n: (0, n)),
            pl.BlockSpec((2, F), lambda n: (0, 0)),
        ],
        out_specs=(
            pl.BlockSpec((BLK,), lambda n: (n,)),
            pl.BlockSpec((BLK,), lambda n: (n,)),
            pl.BlockSpec((BLK,), lambda n: (n,)),
        ),
        out_shape=(
            jax.ShapeDtypeStruct((V,), jnp.float32),
            jax.ShapeDtypeStruct((V,), jnp.float32),
            jax.ShapeDtypeStruct((V,), jnp.float32),
        ),
    )(user_t, st)


def _sc_gather5(u, i, j, zi, zj, qu, qi):
    mesh = plsc.VectorSubcoreMesh(core_axis_name="c", subcore_axis_name="s")
    vec = jax.ShapeDtypeStruct((B,), jnp.float32)

    @pl.kernel(
        out_type=[vec, vec, vec, vec, vec],
        mesh=mesh,
        compiler_params=pltpu.CompilerParams(use_tc_tiling_on_sc=False),
        scratch_types=[
            pltpu.VMEM((BPW,), jnp.int32),
            pltpu.VMEM((BPW,), jnp.int32),
            pltpu.VMEM((BPW,), jnp.int32),
            pltpu.VMEM((BPW,), jnp.float32),
            pltpu.VMEM((BPW,), jnp.float32),
            pltpu.VMEM((BPW,), jnp.float32),
            pltpu.VMEM((BPW,), jnp.float32),
            pltpu.VMEM((BPW,), jnp.float32),
            pltpu.SemaphoreType.DMA,
        ],
    )
    def gather_kernel(u_hbm, i_hbm, j_hbm, zi_hbm, zj_hbm, qu_hbm, qi_hbm,
                      yui_hbm, yuj_hbm, gqu_hbm, gqi_hbm, gqj_hbm,
                      idx_u, idx_i, idx_j, b0, b1, b2, b3, b4, sem):
        wid = lax.axis_index("s") * NC + lax.axis_index("c")
        sl = pl.ds(wid * BPW, BPW)
        pltpu.sync_copy(u_hbm.at[sl], idx_u)
        pltpu.sync_copy(i_hbm.at[sl], idx_i)
        pltpu.sync_copy(j_hbm.at[sl], idx_j)
        cps = [
            pltpu.async_copy(zi_hbm.at[idx_u], b0, sem),
            pltpu.async_copy(zj_hbm.at[idx_u], b1, sem),
            pltpu.async_copy(qu_hbm.at[idx_u], b2, sem),
            pltpu.async_copy(qi_hbm.at[idx_i], b3, sem),
            pltpu.async_copy(qi_hbm.at[idx_j], b4, sem),
        ]
        for cp in cps:
            cp.wait()
        pltpu.sync_copy(b0, yui_hbm.at[sl])
        pltpu.sync_copy(b1, yuj_hbm.at[sl])
        pltpu.sync_copy(b2, gqu_hbm.at[sl])
        pltpu.sync_copy(b3, gqi_hbm.at[sl])
        pltpu.sync_copy(b4, gqj_hbm.at[sl])

    return gather_kernel(u, i, j, zi, zj, qu, qi)


def _tc_loss_body(yui_ref, yuj_ref, gqu_ref, gqi_ref, gqj_ref, loss_ref):
    y_ui = yui_ref[...]
    y_uj = yuj_ref[...]
    reg = REG * (jnp.sum(gqu_ref[...]) + jnp.sum(gqi_ref[...])
                 + jnp.sum(gqj_ref[...]))
    d = y_ui - y_uj
    ls = jnp.minimum(d, 0.0) - jnp.log(1.0 + jnp.exp(-jnp.abs(d)))
    loss_ref[0, 0] = reg - jnp.sum(ls) * _INV_LN2


def _tc_loss(y_ui, y_uj, gqu, gqi, gqj):
    return pl.pallas_call(
        _tc_loss_body,
        out_shape=jax.ShapeDtypeStruct((1, 1), jnp.float32),
        out_specs=pl.BlockSpec(memory_space=pltpu.SMEM),
    )(y_ui, y_uj, gqu, gqi, gqj)


def kernel(u, i, j, embed_user, embed_item):
    user_t = embed_user.T
    item_t = embed_item.T
    ci, cj = _sc_counts(i, j)
    st, q_item = _tc_item(item_t, ci, cj)
    z_ui, z_uj, q_user = _tc_user(user_t, st)
    y_ui, y_uj, gqu, gqi, gqj = _sc_gather5(u, i, j, z_ui, z_uj, q_user, q_item)
    loss2d = _tc_loss(y_ui, y_uj, gqu, gqi, gqj)
    return y_ui, y_uj, loss2d[0, 0]


# speedup vs baseline: 35.9762x; 1.0281x over previous
import functools
import math

import jax
import jax.numpy as jnp
from jax import lax
from jax.experimental import pallas as pl
from jax.experimental.pallas import tpu as pltpu
from jax.experimental.pallas import tpu_sc as plsc

V = 1000000
F = 64
B = 4096
REG = 0.01

NC = 2
NS = 16
NW = NC * NS
BPW = B // NW

BLK = 32768
GRID = (V + BLK - 1) // BLK

ZCH = 16384
WSPAN = V // 8

_INV_LN2 = 1.0 / math.log(2.0)


def _sc_counts(i, j):
    mesh = plsc.VectorSubcoreMesh(core_axis_name="c", subcore_axis_name="s")
    cnt = jax.ShapeDtypeStruct((V,), jnp.float32)

    @pl.kernel(
        out_type=[cnt, cnt],
        mesh=mesh,
        compiler_params=pltpu.CompilerParams(use_tc_tiling_on_sc=False),
        scratch_types=[
            pltpu.VMEM((2, BPW), jnp.int32),
            pltpu.VMEM((BPW,), jnp.float32),
            pltpu.VMEM((ZCH,), jnp.float32),
            pltpu.VMEM_SHARED((V,), jnp.float32),
            pltpu.SemaphoreType.DMA,
        ],
    )
    def counts_kernel(i_hbm, j_hbm, ci_hbm, cj_hbm, idx2, ones, zbuf, csh, sem):
        c = lax.axis_index("c")
        s = lax.axis_index("s")

        @pl.loop(0, BPW, step=16)
        def _(k):
            ones[pl.ds(k, 16)] = jnp.full((16,), 1.0, jnp.float32)

        @pl.loop(0, ZCH, step=16)
        def _(k):
            zbuf[pl.ds(k, 16)] = jnp.zeros((16,), jnp.float32)

        @pl.when(s < 8)
        def _():
            n_full = WSPAN // ZCH
            tail = WSPAN - n_full * ZCH
            cps = []
            for kk in range(n_full):
                cps.append(pltpu.async_copy(
                    zbuf, csh.at[pl.ds(s * WSPAN + kk * ZCH, ZCH)], sem))
            cps.append(pltpu.async_copy(
                zbuf.at[pl.ds(0, tail)],
                csh.at[pl.ds(s * WSPAN + n_full * ZCH, tail)], sem))
            for cp in cps:
                cp.wait()

        plsc.subcore_barrier()

        @pl.when(c == 0)
        def _():
            pltpu.sync_copy(i_hbm.at[pl.ds(s * 256, BPW)], idx2.at[0])
            pltpu.sync_copy(i_hbm.at[pl.ds(s * 256 + BPW, BPW)], idx2.at[1])

        @pl.when(c == 1)
        def _():
            pltpu.sync_copy(j_hbm.at[pl.ds(s * 256, BPW)], idx2.at[0])
            pltpu.sync_copy(j_hbm.at[pl.ds(s * 256 + BPW, BPW)], idx2.at[1])

        pltpu.sync_copy(ones, csh.at[idx2.at[0]], add=True)
        pltpu.sync_copy(ones, csh.at[idx2.at[1]], add=True)

        plsc.subcore_barrier()

        @pl.when(jnp.logical_and(s < 8, c == 0))
        def _():
            pltpu.sync_copy(csh.at[pl.ds(s * WSPAN, WSPAN)],
                            ci_hbm.at[pl.ds(s * WSPAN, WSPAN)])

        @pl.when(jnp.logical_and(s < 8, c == 1))
        def _():
            pltpu.sync_copy(csh.at[pl.ds(s * WSPAN, WSPAN)],
                            cj_hbm.at[pl.ds(s * WSPAN, WSPAN)])

    return counts_kernel(i, j)


def _tc_item_body(it_ref, ci_ref, cj_ref, st_ref, q_ref, acc_ref):
    n = pl.program_id(0)

    @pl.when(n == 0)
    def _():
        acc_ref[...] = jnp.zeros((2, F, 128), jnp.float32)

    e = it_ref[...]
    ci = ci_ref[...]
    cj = cj_ref[...]

    q_ref[...] = jnp.sum(e * e, axis=0)

    def accum(e_, ci_, cj_):
        ai = acc_ref[0]
        aj = acc_ref[1]
        cib = ci_[None, :]
        cjb = cj_[None, :]
        for k in range(BLK // 128):
            s = slice(k * 128, (k + 1) * 128)
            ec = e_[:, s]
            ai = ai + ec * cib[:, s]
            aj = aj + ec * cjb[:, s]
        acc_ref[0] = ai
        acc_ref[1] = aj

    @pl.when(n < GRID - 1)
    def _():
        accum(e, ci, cj)

    @pl.when(n == GRID - 1)
    def _():
        pos = n * BLK + lax.broadcasted_iota(jnp.int32, (BLK,), 0)
        valid = pos < V
        accum(jnp.where(valid[None, :], e, 0.0),
              jnp.where(valid, ci, 0.0), jnp.where(valid, cj, 0.0))
        st_ref[0:1, :] = jnp.sum(acc_ref[0], axis=1)[None, :]
        st_ref[1:2, :] = jnp.sum(acc_ref[1], axis=1)[None, :]


def _tc_item(item_t, ci, cj):
    return pl.pallas_call(
        _tc_item_body,
        grid=(GRID,),
        in_specs=[
            pl.BlockSpec((F, BLK), lambda n: (0, n)),
            pl.BlockSpec((BLK,), lambda n: (n,)),
            pl.BlockSpec((BLK,), lambda n: (n,)),
        ],
        out_specs=(
            pl.BlockSpec((2, F), lambda n: (0, 0)),
            pl.BlockSpec((BLK,), lambda n: (n,)),
        ),
        out_shape=(
            jax.ShapeDtypeStruct((2, F), jnp.float32),
            jax.ShapeDtypeStruct((V,), jnp.float32),
        ),
        scratch_shapes=[pltpu.VMEM((2, F, 128), jnp.float32)],
    )(item_t, ci, cj)


def _tc_user_body(ut_ref, st_ref, zi_ref, zj_ref, q_ref):
    e = ut_ref[...]
    w = st_ref[...]
    z = lax.dot_general(w, e, (((1,), (0,)), ((), ())),
                        precision=lax.Precision.HIGHEST,
                        preferred_element_type=jnp.float32)
    zi_ref[...] = z[0, :]
    zj_ref[...] = z[1, :]
    q_ref[...] = jnp.sum(e * e, axis=0)


def _tc_user(user_t, st):
    return pl.pallas_call(
        _tc_user_body,
        grid=(GRID,),
        in_specs=[
            pl.BlockSpec((F, BLK), lambda n: (0, n)),
            pl.BlockSpec((2, F), lambda n: (0, 0)),
        ],
        out_specs=(
            pl.BlockSpec((BLK,), lambda n: (n,)),
            pl.BlockSpec((BLK,), lambda n: (n,)),
            pl.BlockSpec((BLK,), lambda n: (n,)),
        ),
        out_shape=(
            jax.ShapeDtypeStruct((V,), jnp.float32),
            jax.ShapeDtypeStruct((V,), jnp.float32),
            jax.ShapeDtypeStruct((V,), jnp.float32),
        ),
    )(user_t, st)


def _sc_gather5(u, i, j, zi, zj, qu, qi):
    mesh = plsc.VectorSubcoreMesh(core_axis_name="c", subcore_axis_name="s")
    vec = jax.ShapeDtypeStruct((B,), jnp.float32)

    @pl.kernel(
        out_type=[vec, vec, vec, vec, vec],
        mesh=mesh,
        compiler_params=pltpu.CompilerParams(use_tc_tiling_on_sc=False),
        scratch_types=[
            pltpu.VMEM((BPW,), jnp.int32),
            pltpu.VMEM((BPW,), jnp.int32),
            pltpu.VMEM((BPW,), jnp.int32),
            pltpu.VMEM((BPW,), jnp.float32),
            pltpu.VMEM((BPW,), jnp.float32),
            pltpu.VMEM((BPW,), jnp.float32),
            pltpu.VMEM((BPW,), jnp.float32),
            pltpu.VMEM((BPW,), jnp.float32),
            pltpu.SemaphoreType.DMA,
        ],
    )
    def gather_kernel(u_hbm, i_hbm, j_hbm, zi_hbm, zj_hbm, qu_hbm, qi_hbm,
                      yui_hbm, yuj_hbm, gqu_hbm, gqi_hbm, gqj_hbm,
                      idx_u, idx_i, idx_j, b0, b1, b2, b3, b4, sem):
        wid = lax.axis_index("s") * NC + lax.axis_index("c")
        sl = pl.ds(wid * BPW, BPW)
        pltpu.sync_copy(u_hbm.at[sl], idx_u)
        pltpu.sync_copy(i_hbm.at[sl], idx_i)
        pltpu.sync_copy(j_hbm.at[sl], idx_j)
        cps = [
            pltpu.async_copy(zi_hbm.at[idx_u], b0, sem),
            pltpu.async_copy(zj_hbm.at[idx_u], b1, sem),
            pltpu.async_copy(qu_hbm.at[idx_u], b2, sem),
            pltpu.async_copy(qi_hbm.at[idx_i], b3, sem),
            pltpu.async_copy(qi_hbm.at[idx_j], b4, sem),
        ]
        for cp in cps:
            cp.wait()
        pltpu.sync_copy(b0, yui_hbm.at[sl])
        pltpu.sync_copy(b1, yuj_hbm.at[sl])
        pltpu.sync_copy(b2, gqu_hbm.at[sl])
        pltpu.sync_copy(b3, gqi_hbm.at[sl])
        pltpu.sync_copy(b4, gqj_hbm.at[sl])

    return gather_kernel(u, i, j, zi, zj, qu, qi)


def _tc_loss_body(yui_ref, yuj_ref, gqu_ref, gqi_ref, gqj_ref, loss_ref):
    y_ui = yui_ref[...]
    y_uj = yuj_ref[...]
    reg = REG * (jnp.sum(gqu_ref[...]) + jnp.sum(gqi_ref[...])
                 + jnp.sum(gqj_ref[...]))
    d = y_ui - y_uj
    ls = jnp.minimum(d, 0.0) - jnp.log(1.0 + jnp.exp(-jnp.abs(d)))
    loss_ref[0, 0] = reg - jnp.sum(ls) * _INV_LN2


def _tc_loss(y_ui, y_uj, gqu, gqi, gqj):
    return pl.pallas_call(
        _tc_loss_body,
        out_shape=jax.ShapeDtypeStruct((1, 1), jnp.float32),
        out_specs=pl.BlockSpec(memory_space=pltpu.SMEM),
    )(y_ui, y_uj, gqu, gqi, gqj)


def kernel(u, i, j, embed_user, embed_item):
    user_t = embed_user.T
    item_t = embed_item.T
    ci, cj = _sc_counts(i, j)
    st, q_item = _tc_item(item_t, ci, cj)
    z_ui, z_uj, q_user = _tc_user(user_t, st)
    y_ui, y_uj, gqu, gqi, gqj = _sc_gather5(u, i, j, z_ui, z_uj, q_user, q_item)
    loss2d = _tc_loss(y_ui, y_uj, gqu, gqi, gqj)
    return y_ui, y_uj, loss2d[0, 0]
